# baseline (speedup 1.0000x reference)
import functools

import jax
import jax.numpy as jnp
from jax import lax
from jax.experimental import pallas as pl
from jax.experimental.pallas import tpu as pltpu
from jax.experimental.pallas import tpu_sc as plsc

B = 2
C = 96
H = 384
W = 384
N = H * W
NC = 2
NS = 16
NW = NC * NS
ROWS_PER_W = (B * H) // NW
HALF = N // 2
NTASK = B * (C + 1) * 2
KMAX = (NTASK + NW - 1) // NW
WIN = 4096
NWIN = N // WIN
L = 16


def _pass1_body(flow_hbm, metric_hbm, dst_hbm, wgt_hbm,
                fxb, fyb, mb, dstb, wgtb):
    wid = lax.axis_index("s") * NC + lax.axis_index("c")
    lanes = lax.iota(jnp.int32, L)

    def row_task(r, _):
        row_id = wid * ROWS_PER_W + r
        b = row_id // H
        y = row_id - b * H
        src_off = b * N + y * W
        pltpu.sync_copy(flow_hbm.at[pl.ds((b * 2 + 0) * N + y * W, W)], fxb)
        pltpu.sync_copy(flow_hbm.at[pl.ds((b * 2 + 1) * N + y * W, W)], fyb)
        pltpu.sync_copy(metric_hbm.at[pl.ds(src_off, W)], mb)
        yf = y.astype(jnp.float32)

        def vec(j, _):
            xi = j * L + lanes
            fx = xi.astype(jnp.float32) + fxb[pl.ds(j * L, L)]
            fy = yf + fyb[pl.ds(j * L, L)]
            x0 = fx.astype(jnp.int32)
            x0 = jnp.where(x0.astype(jnp.float32) > fx, x0 - 1, x0)
            y0 = fy.astype(jnp.int32)
            y0 = jnp.where(y0.astype(jnp.float32) > fy, y0 - 1, y0)
            wx1 = fx - x0.astype(jnp.float32)
            wx0 = 1.0 - wx1
            wy1 = fy - y0.astype(jnp.float32)
            wy0 = 1.0 - wy1
            m = jnp.exp(mb[pl.ds(j * L, L)])
            x1 = x0 + 1
            y1 = y0 + 1
            vx0 = (x0 >= 0) & (x0 < W)
            vx1 = (x1 >= 0) & (x1 < W)
            vy0 = (y0 >= 0) & (y0 < H)
            vy1 = (y1 >= 0) & (y1 < H)
            cx0 = jnp.minimum(jnp.maximum(x0, 0), W - 1)
            cx1 = jnp.minimum(jnp.maximum(x1, 0), W - 1)
            cy0 = jnp.minimum(jnp.maximum(y0, 0), H - 1) * W
            cy1 = jnp.minimum(jnp.maximum(y1, 0), H - 1) * W
            zero = jnp.zeros((L,), jnp.float32)
            taps = (
                (cy0 + cx0, jnp.where(vx0 & vy0, wx0 * wy0 * m, zero)),
                (cy0 + cx1, jnp.where(vx1 & vy0, wx1 * wy0 * m, zero)),
                (cy1 + cx0, jnp.where(vx0 & vy1, wx0 * wy1 * m, zero)),
                (cy1 + cx1, jnp.where(vx1 & vy1, wx1 * wy1 * m, zero)),
            )
            for t in range(4):
                dstb[t, pl.ds(j * L, L)] = taps[t][0]
                wgtb[t, pl.ds(j * L, L)] = taps[t][1]
            return 0

        lax.fori_loop(0, W // L, vec, 0)
        for t in range(4):
            off = (b * 4 + t) * N + y * W
            pltpu.sync_copy(dstb.at[t], dst_hbm.at[pl.ds(off, W)])
            pltpu.sync_copy(wgtb.at[t], wgt_hbm.at[pl.ds(off, W)])
        return 0

    lax.fori_loop(0, ROWS_PER_W, row_task, 0)


def _pass2_body(dst_hbm, wgt_hbm, val_hbm, out_hbm,
                acc, d0, d1, d2, d3, w0, w1, w2, w3, vbuf):
    wid = lax.axis_index("s") * NC + lax.axis_index("c")
    dbufs = (d0, d1, d2, d3)
    wbufs = (w0, w1, w2, w3)
    zero16 = jnp.zeros((L,), jnp.float32)

    def task(k, _):
        tid = k * NW + wid

        @pl.when(tid < NTASK)
        def _():
            b = tid // (2 * (C + 1))
            rem = tid - b * (2 * (C + 1))
            ch = rem // 2
            half = rem - ch * 2
            base = half * HALF
            ch_flat = b * (C + 1) + ch

            def zacc(z, _):
                acc[pl.ds(z * L, L)] = zero16
                return 0

            lax.fori_loop(0, HALF // L, zacc, 0)

            def window(wi, _):
                off = wi * WIN
                for t in range(4):
                    toff = (b * 4 + t) * N + off
                    pltpu.sync_copy(dst_hbm.at[pl.ds(toff, WIN)], dbufs[t])
                    pltpu.sync_copy(wgt_hbm.at[pl.ds(toff, WIN)], wbufs[t])
                pltpu.sync_copy(val_hbm.at[pl.ds(ch_flat * N + off, WIN)],
                                vbuf)

                def vec(i, _):
                    v = vbuf[pl.ds(i * L, L)]
                    for t in range(4):
                        d = dbufs[t][pl.ds(i * L, L)]
                        wv = wbufs[t][pl.ds(i * L, L)]
                        local = d - base
                        inb = (local >= 0) & (local < HALF)
                        idx = jnp.where(inb, local, 0)
                        plsc.addupdate_scatter(acc, [idx], wv * v, mask=inb)
                    return 0

                lax.fori_loop(0, WIN // L, vec, 0)
                return 0

            lax.fori_loop(0, NWIN, window, 0)
            pltpu.sync_copy(acc, out_hbm.at[pl.ds(ch_flat * N + base, HALF)])

        return 0

    lax.fori_loop(0, KMAX, task, 0)


def _norm_body(num_ref, den_ref, o_ref):
    o_ref[...] = num_ref[...] / (den_ref[...] + 1e-7)


def kernel(tenInput, tenFlow, tenMetric):
    mesh = plsc.VectorSubcoreMesh(core_axis_name="c", subcore_axis_name="s")

    flow_flat = tenFlow.reshape(B * 2 * N)
    metric_flat = tenMetric.reshape(B * N)
    ones = jnp.ones((B, 1, H, W), dtype=tenInput.dtype)
    val_flat = jnp.concatenate([tenInput, ones], axis=1).reshape(B * (C + 1) * N)

    sc_params = pltpu.CompilerParams(needs_layout_passes=False)
    pass1 = functools.partial(
        pl.kernel,
        mesh=mesh,
        compiler_params=sc_params,
        out_type=(
            jax.ShapeDtypeStruct((B * 4 * N,), jnp.int32),
            jax.ShapeDtypeStruct((B * 4 * N,), jnp.float32),
        ),
        scratch_types=[
            pltpu.VMEM((W,), jnp.float32),
            pltpu.VMEM((W,), jnp.float32),
            pltpu.VMEM((W,), jnp.float32),
            pltpu.VMEM((4, W), jnp.int32),
            pltpu.VMEM((4, W), jnp.float32),
        ],
    )(_pass1_body)
    dst_flat, wgt_flat = pass1(flow_flat, metric_flat)

    pass2 = functools.partial(
        pl.kernel,
        mesh=mesh,
        compiler_params=sc_params,
        out_type=jax.ShapeDtypeStruct((B * (C + 1) * N,), jnp.float32),
        scratch_types=[
            pltpu.VMEM((HALF,), jnp.float32),
            pltpu.VMEM((WIN,), jnp.int32),
            pltpu.VMEM((WIN,), jnp.int32),
            pltpu.VMEM((WIN,), jnp.int32),
            pltpu.VMEM((WIN,), jnp.int32),
            pltpu.VMEM((WIN,), jnp.float32),
            pltpu.VMEM((WIN,), jnp.float32),
            pltpu.VMEM((WIN,), jnp.float32),
            pltpu.VMEM((WIN,), jnp.float32),
            pltpu.VMEM((WIN,), jnp.float32),
        ],
    )(_pass2_body)
    out97 = pass2(dst_flat, wgt_flat, val_flat).reshape(B, C + 1, H, W)

    num = out97[:, :C]
    den = out97[:, C:]
    out = pl.pallas_call(
        _norm_body,
        grid=(B, C),
        in_specs=[
            pl.BlockSpec((1, 1, H, W), lambda b, c: (b, c, 0, 0)),
            pl.BlockSpec((1, 1, H, W), lambda b, c: (b, 0, 0, 0)),
        ],
        out_specs=pl.BlockSpec((1, 1, H, W), lambda b, c: (b, c, 0, 0)),
        out_shape=jax.ShapeDtypeStruct((B, C, H, W), jnp.float32),
    )(num, den)
    return out
